# Initial kernel scaffold; baseline (speedup 1.0000x reference)
import jax
import jax.numpy as jnp
from jax.experimental import pallas as pl


def kernel(x, W):
    raise NotImplementedError("write your pallas kernel here")


# speedup vs baseline: 4.6688x; 4.6688x over previous
import jax
import jax.numpy as jnp
from jax.experimental import pallas as pl


def _broadcast_body(w_ref, o_ref):
    o_ref[...] = jnp.broadcast_to(w_ref[...][None, :, :], o_ref.shape)


def kernel(x, W):
    B = x.shape[0]
    T, H = W.shape
    BT = 256
    return pl.pallas_call(
        _broadcast_body,
        grid=(T // BT,),
        in_specs=[pl.BlockSpec((BT, H), lambda i: (i, 0))],
        out_specs=pl.BlockSpec((B, BT, H), lambda i: (0, i, 0)),
        out_shape=jax.ShapeDtypeStruct((B, T, H), W.dtype),
    )(W)


# speedup vs baseline: 5.0375x; 1.0790x over previous
import jax
import jax.numpy as jnp
from jax.experimental import pallas as pl


def _broadcast_body(w_ref, o_ref):
    o_ref[...] = jnp.broadcast_to(w_ref[...][None, :, :], o_ref.shape)


def kernel(x, W):
    B = x.shape[0]
    T, H = W.shape
    BT = 512
    return pl.pallas_call(
        _broadcast_body,
        grid=(T // BT,),
        in_specs=[pl.BlockSpec((BT, H), lambda i: (i, 0))],
        out_specs=pl.BlockSpec((B, BT, H), lambda i: (0, i, 0)),
        out_shape=jax.ShapeDtypeStruct((B, T, H), W.dtype),
    )(W)
